# Initial kernel scaffold; baseline (speedup 1.0000x reference)
import jax
import jax.numpy as jnp
from jax.experimental import pallas as pl


def kernel(ego_embeddings, a_in_indices, a_in_values, a_in_plusI_indices, a_in_plusI_values, W1, b1, W2, b2):
    raise NotImplementedError("write your pallas kernel here")


# speedup vs baseline: 2.9580x; 2.9580x over previous
import functools

import jax
import jax.numpy as jnp
from jax import lax
from jax.experimental import pallas as pl
from jax.experimental.pallas import tpu as pltpu
from jax.experimental.pallas import tpu_sc as plsc

N = 100000
D = 64
DH = 32
NC = 2
NS = 16
RPC = N // NC
STRIPE = 3128
WMAIN = 3080
WEXTRA = STRIPE - WMAIN
RGARB = NS * STRIPE
ACC_ROWS = RGARB + 8
K = 512
SUB = 128
NSUB = K // SUB
ZREM = STRIPE - (STRIPE // SUB) * SUB


def _ceil_batches(e):
    return -(-e // (NS * K))


def _pad_edges(rows, cols, vals, e_pad):
    e = rows.shape[0]
    pad = e_pad - e
    rows = jnp.concatenate([rows, jnp.full((pad,), N, jnp.int32)])
    cols = jnp.concatenate([cols, jnp.zeros((pad,), jnp.int32)])
    vals = jnp.concatenate([vals, jnp.zeros((pad,), jnp.float32)])
    return rows, cols, vals


def _sc_body(nb_a, nb_b,
             ego_lo, ego_hi, r_a, c_a, v_a, r_b, c_b, v_b,
             o_s_lo, o_s_hi, o_li_lo, o_li_hi,
             acc, gbuf, rowbuf, colbuf, valbuf, destbuf, sem):
    c = lax.axis_index("c")
    s = lax.axis_index("s")
    lo = c * RPC
    hi = lo + RPC

    zero16 = jnp.zeros((16,), jnp.float32)

    def one_pass(rows_h, cols_h, vals_h, ego_h, out_h, nbatches):
        def _zb(r, carry):
            gbuf[0, r, pl.ds(0, 16)] = zero16
            gbuf[0, r, pl.ds(16, 16)] = zero16
            return carry
        lax.fori_loop(0, SUB, _zb, 0)

        def zeroq(q, carry):
            pltpu.sync_copy(gbuf.at[0],
                            acc.at[pl.ds(s * STRIPE + q * SUB, SUB)])
            return carry
        lax.fori_loop(0, STRIPE // SUB, zeroq, 0)
        pltpu.sync_copy(gbuf.at[0, pl.ds(0, ZREM)],
                        acc.at[pl.ds(s * STRIPE + (STRIPE // SUB) * SUB, ZREM)])
        plsc.subcore_barrier()

        tile_base = s * (nbatches * K)

        def batch(b, carry):
            base = tile_base + b * K
            pltpu.sync_copy(rows_h.at[pl.ds(base, K)], rowbuf)
            pltpu.sync_copy(cols_h.at[pl.ds(base, K)], colbuf)
            pltpu.sync_copy(vals_h.at[pl.ds(base, K)], valbuf)

            for i in range(NSUB):
                def prep(j, cc, i=i):
                    off = i * SUB + j * 16
                    r = rowbuf[pl.ds(off, 16)]
                    v = valbuf[pl.ds(off, 16)]
                    m = (r >= lo) & (r < hi)
                    destbuf[i, pl.ds(j * 16, 16)] = jnp.where(m, r - lo, RGARB)
                    valbuf[pl.ds(off, 16)] = jnp.where(m, v, 0.0)
                    return cc
                lax.fori_loop(0, SUB // 16, prep, 0)

            cps = [
                pltpu.async_copy(ego_h.at[colbuf.at[pl.ds(i * SUB, SUB)]],
                                 gbuf.at[i], sem)
                for i in range(NSUB)
            ]
            for cp in cps:
                cp.wait()

            for i in range(NSUB):
                def scale(j, cc, i=i):
                    vk = valbuf[pl.ds(i * SUB + j * 16, 16)]
                    for k in range(16):
                        e = j * 16 + k
                        sv = vk[k]
                        gbuf[i, e, pl.ds(0, 16)] = gbuf[i, e, pl.ds(0, 16)] * sv
                        gbuf[i, e, pl.ds(16, 16)] = gbuf[i, e, pl.ds(16, 16)] * sv
                    return cc
                lax.fori_loop(0, SUB // 16, scale, 0)

            for i in range(NSUB):
                pltpu.sync_copy(gbuf.at[i], acc.at[destbuf.at[i]], add=True)
            return carry

        lax.fori_loop(0, nbatches, batch, 0)
        plsc.subcore_barrier()

        pltpu.sync_copy(acc.at[pl.ds(s * STRIPE, WMAIN)],
                        out_h.at[pl.ds(lo + s * STRIPE, WMAIN)])

        @pl.when(s < NS - 1)
        def _():
            pltpu.sync_copy(acc.at[pl.ds(s * STRIPE + WMAIN, WEXTRA)],
                            out_h.at[pl.ds(lo + s * STRIPE + WMAIN, WEXTRA)])

        plsc.subcore_barrier()

    one_pass(r_a, c_a, v_a, ego_lo, o_s_lo, nb_a)
    one_pass(r_a, c_a, v_a, ego_hi, o_s_hi, nb_a)
    one_pass(r_b, c_b, v_b, ego_lo, o_li_lo, nb_b)
    one_pass(r_b, c_b, v_b, ego_hi, o_li_hi, nb_b)


def _sc_spmm(ego_lo, ego_hi, r_a, c_a, v_a, r_b, c_b, v_b, nb_a, nb_b):
    mesh = plsc.VectorSubcoreMesh(core_axis_name="c", subcore_axis_name="s",
                                  num_cores=NC, num_subcores=NS)
    out = jax.ShapeDtypeStruct((N, DH), jnp.float32)
    f = pl.kernel(
        functools.partial(_sc_body, nb_a, nb_b),
        out_type=(out, out, out, out),
        mesh=mesh,
        scratch_types=[
            pltpu.VMEM_SHARED((ACC_ROWS, DH), jnp.float32),
            pltpu.VMEM((NSUB, SUB, DH), jnp.float32),
            pltpu.VMEM((K,), jnp.int32),
            pltpu.VMEM((K,), jnp.int32),
            pltpu.VMEM((K,), jnp.float32),
            pltpu.VMEM((NSUB, SUB), jnp.int32),
            pltpu.SemaphoreType.DMA,
        ],
        compiler_params=pltpu.CompilerParams(use_tc_tiling_on_sc=False),
        name="sc_coo_spmm",
    )
    return f(ego_lo, ego_hi, r_a, c_a, v_a, r_b, c_b, v_b)


def _tc_body(sl, sh, ll, lh, ego, w1, b1, w2, b2, out):
    xli = jnp.concatenate([ll[...], lh[...]], axis=1)
    xint = jnp.concatenate([sl[...], sh[...]], axis=1) * ego[...]
    y = (lax.dot_general(xli, w1[...], (((1,), (1,)), ((), ())),
                         preferred_element_type=jnp.float32)
         + lax.dot_general(xint, w2[...], (((1,), (1,)), ((), ())),
                           preferred_element_type=jnp.float32)
         + b1[...] + b2[...])
    out[...] = jnp.where(y >= 0, y, 0.01 * y)


def _tc_dense(s_lo, s_hi, li_lo, li_hi, ego, W1, b1, W2, b2):
    BR = 1000
    grid = (N // BR,)
    half = pl.BlockSpec((BR, DH), lambda i: (i, 0))
    full = pl.BlockSpec((BR, D), lambda i: (i, 0))
    wspec = pl.BlockSpec((D, D), lambda i: (0, 0))
    bspec = pl.BlockSpec((1, D), lambda i: (0, 0))
    return pl.pallas_call(
        _tc_body,
        grid=grid,
        in_specs=[half, half, half, half, full, wspec, bspec, wspec, bspec],
        out_specs=full,
        out_shape=jax.ShapeDtypeStruct((N, D), jnp.float32),
    )(s_lo, s_hi, li_lo, li_hi, ego,
      W1, b1.reshape(1, D), W2, b2.reshape(1, D))


def kernel(ego_embeddings, a_in_indices, a_in_values, a_in_plusI_indices,
           a_in_plusI_values, W1, b1, W2, b2):
    ego_lo = ego_embeddings[:, :DH]
    ego_hi = ego_embeddings[:, DH:]

    nb_a = _ceil_batches(a_in_values.shape[0])
    nb_b = _ceil_batches(a_in_plusI_values.shape[0])
    r_a, c_a, v_a = _pad_edges(a_in_indices[0], a_in_indices[1], a_in_values,
                               nb_a * NS * K)
    r_b, c_b, v_b = _pad_edges(a_in_plusI_indices[0], a_in_plusI_indices[1],
                               a_in_plusI_values, nb_b * NS * K)

    s_lo, s_hi, li_lo, li_hi = _sc_spmm(ego_lo, ego_hi,
                                        r_a, c_a, v_a, r_b, c_b, v_b,
                                        nb_a, nb_b)
    return _tc_dense(s_lo, s_hi, li_lo, li_hi, ego_embeddings, W1, b1, W2, b2)


# speedup vs baseline: 3.0070x; 1.0166x over previous
import functools

import jax
import jax.numpy as jnp
from jax import lax
from jax.experimental import pallas as pl
from jax.experimental.pallas import tpu as pltpu
from jax.experimental.pallas import tpu_sc as plsc

N = 100000
D = 64
DH = 32
NC = 2
NS = 16
RPC = N // NC
STRIPE = 3128
WMAIN = 3080
WEXTRA = STRIPE - WMAIN
RGARB = NS * STRIPE
ACC_ROWS = RGARB + 8
K = 768
SUB = 128
NSUB = K // SUB
ZREM = STRIPE - (STRIPE // SUB) * SUB


def _ceil_batches(e):
    nb = -(-e // (NS * K))
    return nb + (nb % 2)


def _pad_edges(rows, cols, vals, e_pad):
    e = rows.shape[0]
    pad = e_pad - e
    rows = jnp.concatenate([rows, jnp.full((pad,), N, jnp.int32)])
    cols = jnp.concatenate([cols, jnp.zeros((pad,), jnp.int32)])
    vals = jnp.concatenate([vals, jnp.zeros((pad,), jnp.float32)])
    return rows, cols, vals


def _sc_body(nb_a, nb_b,
             ego_lo, ego_hi, r_a, c_a, v_a, r_b, c_b, v_b,
             o_s_lo, o_s_hi, o_li_lo, o_li_hi,
             acc, gbuf,
             rowbuf0, colbuf0, valbuf0, destbuf0,
             rowbuf1, colbuf1, valbuf1, destbuf1,
             isem, ssem, gsems):
    c = lax.axis_index("c")
    s = lax.axis_index("s")
    lo = c * RPC
    hi = lo + RPC

    zero16 = jnp.zeros((16,), jnp.float32)
    garb16 = jnp.full((16,), RGARB, jnp.int32)
    bufs = ((rowbuf0, colbuf0, valbuf0, destbuf0),
            (rowbuf1, colbuf1, valbuf1, destbuf1))

    for db in (destbuf0, destbuf1):
        for i in range(NSUB):
            def _gi(j, cc, db=db, i=i):
                db[i, pl.ds(j * 16, 16)] = garb16
                return cc
            lax.fori_loop(0, SUB // 16, _gi, 0)

    def _fire_idx(rows_h, cols_h, vals_h, base, rb, cb, vb):
        pltpu.async_copy(rows_h.at[pl.ds(base, K)], rb, isem)
        pltpu.async_copy(cols_h.at[pl.ds(base, K)], cb, isem)
        pltpu.async_copy(vals_h.at[pl.ds(base, K)], vb, isem)

    def _wait_idx(rows_h, vals_h, rb, cb, vb):
        pltpu.make_async_copy(rows_h.at[pl.ds(0, K)], rb, isem).wait()
        pltpu.make_async_copy(rows_h.at[pl.ds(0, K)], cb, isem).wait()
        pltpu.make_async_copy(vals_h.at[pl.ds(0, K)], vb, isem).wait()

    def _wait_scatters(db):
        for i in range(NSUB):
            pltpu.make_async_copy(gbuf.at[i], acc.at[db.at[i]], ssem).wait()

    def one_pass(rows_h, cols_h, vals_h, ego_h, out_h, nbatches):
        for i in range(NSUB):
            def _zb(r, cc, i=i):
                gbuf[i, r, pl.ds(0, 16)] = zero16
                gbuf[i, r, pl.ds(16, 16)] = zero16
                return cc
            lax.fori_loop(0, SUB, _zb, 0)

        def zeroq(q, carry):
            pltpu.sync_copy(gbuf.at[0],
                            acc.at[pl.ds(s * STRIPE + q * SUB, SUB)])
            return carry
        lax.fori_loop(0, STRIPE // SUB, zeroq, 0)
        pltpu.sync_copy(gbuf.at[0, pl.ds(0, ZREM)],
                        acc.at[pl.ds(s * STRIPE + (STRIPE // SUB) * SUB, ZREM)])
        plsc.subcore_barrier()

        tile_base = s * (nbatches * K)

        for i in range(NSUB):
            pltpu.async_copy(gbuf.at[i], acc.at[destbuf0.at[i]], ssem,
                             add=True)
        _fire_idx(rows_h, cols_h, vals_h, tile_base,
                  rowbuf0, colbuf0, valbuf0)

        def batch(b, cur, nxt):
            rb, cb, vb, db = cur
            nrb, ncb, nvb, ndb = nxt
            _wait_idx(rows_h, vals_h, rb, cb, vb)
            bn = jnp.minimum(b + 1, nbatches - 1)
            _fire_idx(rows_h, cols_h, vals_h, tile_base + bn * K,
                      nrb, ncb, nvb)

            for i in range(NSUB):
                def prep(j, cc, i=i):
                    off = i * SUB + j * 16
                    r = rb[pl.ds(off, 16)]
                    v = vb[pl.ds(off, 16)]
                    m = (r >= lo) & (r < hi)
                    db[i, pl.ds(j * 16, 16)] = jnp.where(m, r - lo, RGARB)
                    vb[pl.ds(off, 16)] = jnp.where(m, v, 0.0)
                    return cc
                lax.fori_loop(0, SUB // 16, prep, 0)

            _wait_scatters(ndb)

            cps = [
                pltpu.async_copy(ego_h.at[cb.at[pl.ds(i * SUB, SUB)]],
                                 gbuf.at[i], gsems.at[i])
                for i in range(NSUB)
            ]
            for i in range(NSUB):
                cps[i].wait()

                def scale(j, cc, i=i):
                    vk = vb[pl.ds(i * SUB + j * 16, 16)]
                    for k in range(16):
                        e = j * 16 + k
                        sv = vk[k]
                        gbuf[i, e, pl.ds(0, 16)] = gbuf[i, e, pl.ds(0, 16)] * sv
                        gbuf[i, e, pl.ds(16, 16)] = gbuf[i, e, pl.ds(16, 16)] * sv
                    return cc
                lax.fori_loop(0, SUB // 16, scale, 0)
                pltpu.async_copy(gbuf.at[i], acc.at[db.at[i]], ssem, add=True)

        def pair(p, carry):
            batch(2 * p, bufs[0], bufs[1])
            batch(2 * p + 1, bufs[1], bufs[0])
            return carry

        lax.fori_loop(0, nbatches // 2, pair, 0)
        _wait_scatters(destbuf1)
        _wait_idx(rows_h, vals_h, rowbuf0, colbuf0, valbuf0)
        plsc.subcore_barrier()

        pltpu.sync_copy(acc.at[pl.ds(s * STRIPE, WMAIN)],
                        out_h.at[pl.ds(lo + s * STRIPE, WMAIN)])

        @pl.when(s < NS - 1)
        def _():
            pltpu.sync_copy(acc.at[pl.ds(s * STRIPE + WMAIN, WEXTRA)],
                            out_h.at[pl.ds(lo + s * STRIPE + WMAIN, WEXTRA)])

        plsc.subcore_barrier()

    one_pass(r_a, c_a, v_a, ego_lo, o_s_lo, nb_a)
    one_pass(r_a, c_a, v_a, ego_hi, o_s_hi, nb_a)
    one_pass(r_b, c_b, v_b, ego_lo, o_li_lo, nb_b)
    one_pass(r_b, c_b, v_b, ego_hi, o_li_hi, nb_b)


def _sc_spmm(ego_lo, ego_hi, r_a, c_a, v_a, r_b, c_b, v_b, nb_a, nb_b):
    mesh = plsc.VectorSubcoreMesh(core_axis_name="c", subcore_axis_name="s",
                                  num_cores=NC, num_subcores=NS)
    out = jax.ShapeDtypeStruct((N, DH), jnp.float32)
    f = pl.kernel(
        functools.partial(_sc_body, nb_a, nb_b),
        out_type=(out, out, out, out),
        mesh=mesh,
        scratch_types=[
            pltpu.VMEM_SHARED((ACC_ROWS, DH), jnp.float32),
            pltpu.VMEM((NSUB, SUB, DH), jnp.float32),
            pltpu.VMEM((K,), jnp.int32),
            pltpu.VMEM((K,), jnp.int32),
            pltpu.VMEM((K,), jnp.float32),
            pltpu.VMEM((NSUB, SUB), jnp.int32),
            pltpu.VMEM((K,), jnp.int32),
            pltpu.VMEM((K,), jnp.int32),
            pltpu.VMEM((K,), jnp.float32),
            pltpu.VMEM((NSUB, SUB), jnp.int32),
            pltpu.SemaphoreType.DMA,
            pltpu.SemaphoreType.DMA,
            pltpu.SemaphoreType.DMA((NSUB,)),
        ],
        compiler_params=pltpu.CompilerParams(use_tc_tiling_on_sc=False),
        name="sc_coo_spmm",
    )
    return f(ego_lo, ego_hi, r_a, c_a, v_a, r_b, c_b, v_b)


def _tc_body(sl, sh, ll, lh, ego, w1, b1, w2, b2, out):
    xli = jnp.concatenate([ll[...], lh[...]], axis=1)
    xint = jnp.concatenate([sl[...], sh[...]], axis=1) * ego[...]
    y = (lax.dot_general(xli, w1[...], (((1,), (1,)), ((), ())),
                         preferred_element_type=jnp.float32)
         + lax.dot_general(xint, w2[...], (((1,), (1,)), ((), ())),
                           preferred_element_type=jnp.float32)
         + b1[...] + b2[...])
    out[...] = jnp.where(y >= 0, y, 0.01 * y)


def _tc_dense(s_lo, s_hi, li_lo, li_hi, ego, W1, b1, W2, b2):
    BR = 1000
    grid = (N // BR,)
    half = pl.BlockSpec((BR, DH), lambda i: (i, 0))
    full = pl.BlockSpec((BR, D), lambda i: (i, 0))
    wspec = pl.BlockSpec((D, D), lambda i: (0, 0))
    bspec = pl.BlockSpec((1, D), lambda i: (0, 0))
    return pl.pallas_call(
        _tc_body,
        grid=grid,
        in_specs=[half, half, half, half, full, wspec, bspec, wspec, bspec],
        out_specs=full,
        out_shape=jax.ShapeDtypeStruct((N, D), jnp.float32),
    )(s_lo, s_hi, li_lo, li_hi, ego,
      W1, b1.reshape(1, D), W2, b2.reshape(1, D))


def kernel(ego_embeddings, a_in_indices, a_in_values, a_in_plusI_indices,
           a_in_plusI_values, W1, b1, W2, b2):
    ego_lo = ego_embeddings[:, :DH]
    ego_hi = ego_embeddings[:, DH:]

    nb_a = _ceil_batches(a_in_values.shape[0])
    nb_b = _ceil_batches(a_in_plusI_values.shape[0])
    r_a, c_a, v_a = _pad_edges(a_in_indices[0], a_in_indices[1], a_in_values,
                               nb_a * NS * K)
    r_b, c_b, v_b = _pad_edges(a_in_plusI_indices[0], a_in_plusI_indices[1],
                               a_in_plusI_values, nb_b * NS * K)

    s_lo, s_hi, li_lo, li_hi = _sc_spmm(ego_lo, ego_hi,
                                        r_a, c_a, v_a, r_b, c_b, v_b,
                                        nb_a, nb_b)
    return _tc_dense(s_lo, s_hi, li_lo, li_hi, ego_embeddings, W1, b1, W2, b2)
